# baseline (speedup 1.0000x reference)
import functools

import jax
import jax.numpy as jnp
import numpy as np
from jax import lax
from jax.experimental import pallas as pl
from jax.experimental.pallas import tpu as pltpu
from jax.experimental.pallas import tpu_sc as plsc

DIM = 128
MAX_SEQ_LEN = 8192
THETA = 10000.0


@functools.lru_cache(maxsize=1)
def _half_table():
    inv_freq = (1.0 / (THETA ** (np.arange(0, DIM, 2, dtype=np.float32) / DIM))).astype(np.float32)
    t = np.arange(MAX_SEQ_LEN, dtype=np.float32)
    freqs = np.outer(t, inv_freq).astype(np.float32)
    c = np.cos(freqs).astype(np.float32)
    s = np.sin(freqs).astype(np.float32)
    return np.concatenate((c, s), axis=-1)


def _combined_table(position_ids):
    gate = jnp.minimum(position_ids[0, 0], 0).astype(jnp.float32)
    half = jnp.asarray(_half_table()) + gate
    c, s = half[:, :64], half[:, 64:]
    return jnp.concatenate((c, c, s, s), axis=-1)


def _make_gather(batch_total):
    info = plsc.get_sparse_core_info()
    nw = info.num_cores * info.num_subcores
    b_per_w = batch_total // nw
    chunk = 64
    n_chunks = b_per_w // chunk
    nbuf = 6
    ahead = 4

    mesh = plsc.VectorSubcoreMesh(core_axis_name="c", subcore_axis_name="s")

    @functools.partial(
        pl.kernel,
        mesh=mesh,
        out_type=[
            jax.ShapeDtypeStruct((batch_total, DIM), jnp.float32),
            jax.ShapeDtypeStruct((batch_total, DIM), jnp.float32),
        ],
        scratch_types=[
            pltpu.VMEM((b_per_w,), jnp.int32),
            pltpu.VMEM((nbuf, chunk, 2 * DIM), jnp.float32),
        ]
        + [pltpu.SemaphoreType.DMA] * (2 * nbuf),
    )
    def gather_kernel(tab_hbm, idx_hbm, cos_out, sin_out, idx_v, buf, *sems):
        gsems = sems[:nbuf]
        wsems = sems[nbuf:]
        wid = lax.axis_index("s") * info.num_cores + lax.axis_index("c")
        base = wid * b_per_w
        pltpu.sync_copy(idx_hbm.at[pl.ds(base, b_per_w)], idx_v)

        def gather(c):
            p = c % nbuf
            return pltpu.async_copy(
                tab_hbm.at[idx_v.at[pl.ds(c * chunk, chunk)]], buf.at[p], gsems[p])

        def writes(c):
            p = c % nbuf
            rows = pl.ds(base + c * chunk, chunk)
            return [
                pltpu.async_copy(buf.at[p, :, pl.ds(0, DIM)], cos_out.at[rows], wsems[p]),
                pltpu.async_copy(buf.at[p, :, pl.ds(DIM, DIM)], sin_out.at[rows], wsems[p]),
            ]

        g = {c: gather(c) for c in range(min(ahead, n_chunks))}
        w = {}
        for c in range(n_chunks):
            g.pop(c).wait()
            w[c] = writes(c)
            nxt = c + ahead
            if nxt < n_chunks:
                prev = nxt - nbuf
                if prev >= 0:
                    for x in w.pop(prev):
                        x.wait()
                g[nxt] = gather(nxt)
        for c in sorted(w):
            for x in w[c]:
                x.wait()

    return gather_kernel


def kernel(x, seq_len, position_ids):
    del x, seq_len
    tab = _combined_table(position_ids)
    b, s = position_ids.shape
    idx = position_ids.reshape(b * s).astype(jnp.int32)
    cos, sin = _make_gather(b * s)(tab, idx)
    return cos.reshape(b, s, DIM), sin.reshape(b, s, DIM)


# speedup vs baseline: 1.1156x; 1.1156x over previous
import functools

import jax
import jax.numpy as jnp
import numpy as np
from jax import lax
from jax.experimental import pallas as pl
from jax.experimental.pallas import tpu as pltpu
from jax.experimental.pallas import tpu_sc as plsc

DIM = 128
MAX_SEQ_LEN = 8192
THETA = 10000.0


@functools.lru_cache(maxsize=1)
def _combined_table():
    inv_freq = (1.0 / (THETA ** (np.arange(0, DIM, 2, dtype=np.float32) / DIM))).astype(np.float32)
    t = np.arange(MAX_SEQ_LEN, dtype=np.float32)
    freqs = np.outer(t, inv_freq).astype(np.float32)
    c = np.cos(freqs).astype(np.float32)
    s = np.sin(freqs).astype(np.float32)
    return np.concatenate((c, c, s, s), axis=-1)


def _make_gather(batch_total):
    info = plsc.get_sparse_core_info()
    nw = info.num_cores * info.num_subcores
    b_per_w = batch_total // nw
    chunk = 64
    n_chunks = b_per_w // chunk
    nbuf = 7
    ahead = 5

    mesh = plsc.VectorSubcoreMesh(core_axis_name="c", subcore_axis_name="s")

    @functools.partial(
        pl.kernel,
        mesh=mesh,
        out_type=[
            jax.ShapeDtypeStruct((batch_total, DIM), jnp.float32),
            jax.ShapeDtypeStruct((batch_total, DIM), jnp.float32),
        ],
        scratch_types=[
            pltpu.VMEM((b_per_w,), jnp.int32),
            pltpu.VMEM((nbuf, chunk, 2 * DIM), jnp.float32),
        ]
        + [pltpu.SemaphoreType.DMA] * (2 * nbuf),
    )
    def gather_kernel(tab_hbm, idx_hbm, cos_out, sin_out, idx_v, buf, *sems):
        gsems = sems[:nbuf]
        wsems = sems[nbuf:]
        wid = lax.axis_index("s") * info.num_cores + lax.axis_index("c")
        base = wid * b_per_w
        pltpu.sync_copy(idx_hbm.at[pl.ds(base, b_per_w)], idx_v)

        def gather(c):
            p = c % nbuf
            return pltpu.async_copy(
                tab_hbm.at[idx_v.at[pl.ds(c * chunk, chunk)]], buf.at[p], gsems[p])

        def writes(c):
            p = c % nbuf
            rows = pl.ds(base + c * chunk, chunk)
            return [
                pltpu.async_copy(buf.at[p, :, pl.ds(0, DIM)], cos_out.at[rows], wsems[p]),
                pltpu.async_copy(buf.at[p, :, pl.ds(DIM, DIM)], sin_out.at[rows], wsems[p]),
            ]

        g = {c: gather(c) for c in range(min(ahead, n_chunks))}
        w = {}
        for c in range(n_chunks):
            g.pop(c).wait()
            w[c] = writes(c)
            nxt = c + ahead
            if nxt < n_chunks:
                prev = nxt - nbuf
                if prev >= 0:
                    for x in w.pop(prev):
                        x.wait()
                g[nxt] = gather(nxt)
        for c in sorted(w):
            for x in w[c]:
                x.wait()

    return gather_kernel


def kernel(x, seq_len, position_ids):
    del x, seq_len
    tab = jnp.asarray(_combined_table())
    b, s = position_ids.shape
    idx = position_ids.reshape(b * s).astype(jnp.int32)
    cos, sin = _make_gather(b * s)(tab, idx)
    return cos.reshape(b, s, DIM), sin.reshape(b, s, DIM)
